# Initial kernel scaffold; baseline (speedup 1.0000x reference)
import jax
import jax.numpy as jnp
from jax.experimental import pallas as pl


def kernel(x):
    raise NotImplementedError("write your pallas kernel here")


# speedup vs baseline: 9.7599x; 9.7599x over previous
import jax
import jax.numpy as jnp
from jax.experimental import pallas as pl

K = 256
ROW_BLOCK = 8


def _body(x_ref, o_ref):
    x = x_ref[...]
    z = jnp.maximum(jax.lax.bitcast_convert_type(x, jnp.int32), 0)

    def step(i, prefix):
        bit = jnp.int32(1) << (30 - i)
        cand = prefix | bit
        cnt = jnp.sum((z >= cand).astype(jnp.int32), axis=1, keepdims=True)
        return jnp.where(cnt >= K, cand, prefix)

    thresh = jax.lax.fori_loop(0, 31, step, jnp.zeros((x.shape[0], 1), jnp.int32))
    y = jax.lax.bitcast_convert_type(z, jnp.float32)
    o_ref[...] = jnp.where(z >= thresh, y, 0.0)


def kernel(x):
    m, n = x.shape
    grid = (m // ROW_BLOCK,)
    return pl.pallas_call(
        _body,
        grid=grid,
        in_specs=[pl.BlockSpec((ROW_BLOCK, n), lambda i: (i, 0))],
        out_specs=pl.BlockSpec((ROW_BLOCK, n), lambda i: (i, 0)),
        out_shape=jax.ShapeDtypeStruct((m, n), jnp.float32),
    )(x)
